# baseline (speedup 1.0000x reference)
import jax
from jax.experimental import pallas as pl
from jax.experimental.pallas import tpu as pltpu

_ROWS = 768
_COLS = 224 * 224
_NCHUNKS = 16
_CHUNK = _ROWS // _NCHUNKS


def _memcpy_kernel(x_ref, o_ref, sems):
    for k in range(_NCHUNKS):
        pltpu.make_async_copy(
            x_ref.at[pl.ds(k * _CHUNK, _CHUNK)],
            o_ref.at[pl.ds(k * _CHUNK, _CHUNK)],
            sems.at[k],
        ).start()
    for k in range(_NCHUNKS):
        pltpu.make_async_copy(
            x_ref.at[pl.ds(k * _CHUNK, _CHUNK)],
            o_ref.at[pl.ds(k * _CHUNK, _CHUNK)],
            sems.at[k],
        ).wait()


def kernel(input):
    x = input.reshape(_ROWS, _COLS)
    out = pl.pallas_call(
        _memcpy_kernel,
        in_specs=[pl.BlockSpec(memory_space=pl.ANY)],
        out_specs=pl.BlockSpec(memory_space=pl.ANY),
        out_shape=jax.ShapeDtypeStruct((_ROWS, _COLS), x.dtype),
        scratch_shapes=[pltpu.SemaphoreType.DMA((_NCHUNKS,))],
    )(x)
    return out.reshape(input.shape)


# speedup vs baseline: 13.6461x; 13.6461x over previous
import jax
from jax.experimental import pallas as pl
from jax.experimental.pallas import tpu as pltpu

_ROWS = 768
_COLS = 224 * 224
_BR = 16
_N = _ROWS // _BR
_NBUF = 8
_LAG = 4


def _copy_pipeline(x_ref, o_ref, buf, in_sems, out_sems):
    def start_read(n):
        pltpu.make_async_copy(
            x_ref.at[pl.ds(n * _BR, _BR)], buf.at[n % _NBUF],
            in_sems.at[n % _NBUF]).start()

    def wait_read(n):
        pltpu.make_async_copy(
            x_ref.at[pl.ds(n * _BR, _BR)], buf.at[n % _NBUF],
            in_sems.at[n % _NBUF]).wait()

    def start_write(n):
        pltpu.make_async_copy(
            buf.at[n % _NBUF], o_ref.at[pl.ds(n * _BR, _BR)],
            out_sems.at[n % _NBUF]).start()

    def wait_write(n):
        pltpu.make_async_copy(
            buf.at[n % _NBUF], o_ref.at[pl.ds(n * _BR, _BR)],
            out_sems.at[n % _NBUF]).wait()

    for n in range(_N):
        if n >= _NBUF:
            wait_write(n - _NBUF)
        start_read(n)
        if n >= _LAG:
            wait_read(n - _LAG)
            start_write(n - _LAG)
    for n in range(_N - _LAG, _N):
        wait_read(n)
        start_write(n)
    for n in range(_N - _NBUF, _N):
        wait_write(n)


def kernel(input):
    x = input.reshape(_ROWS, _COLS)
    out = pl.pallas_call(
        _copy_pipeline,
        in_specs=[pl.BlockSpec(memory_space=pl.ANY)],
        out_specs=pl.BlockSpec(memory_space=pl.ANY),
        out_shape=jax.ShapeDtypeStruct((_ROWS, _COLS), x.dtype),
        scratch_shapes=[
            pltpu.VMEM((_NBUF, _BR, _COLS), x.dtype),
            pltpu.SemaphoreType.DMA((_NBUF,)),
            pltpu.SemaphoreType.DMA((_NBUF,)),
        ],
    )(x)
    return out.reshape(input.shape)


# speedup vs baseline: 52.1046x; 3.8183x over previous
import jax
from jax.experimental import pallas as pl
from jax.experimental.pallas import tpu as pltpu

_ROWS = 224 * 224
_COLS = 768
_BLOCK_ROWS = 3584


def _copy_block(x_ref, o_ref):
    o_ref[...] = x_ref[...]


def kernel(input):
    x = input.reshape(_COLS, _ROWS).T
    out = pl.pallas_call(
        _copy_block,
        grid=(_ROWS // _BLOCK_ROWS,),
        in_specs=[pl.BlockSpec((_BLOCK_ROWS, _COLS), lambda i: (i, 0))],
        out_specs=pl.BlockSpec((_BLOCK_ROWS, _COLS), lambda i: (i, 0)),
        out_shape=jax.ShapeDtypeStruct((_ROWS, _COLS), x.dtype),
    )(x)
    return out.T.reshape(input.shape)
